# baseline (speedup 1.0000x reference)
import jax
import jax.numpy as jnp
from jax.experimental import pallas as pl
from jax.experimental.pallas import tpu as pltpu

NSTREAMS = 2
BLOCK_M = 512


def _router_block(*refs):
    h_refs = refs[:NSTREAMS]
    w_ref = refs[NSTREAMS]
    out_ref = refs[NSTREAMS + 1]
    w = w_ref[...]

    def probs(h):
        logits = jax.lax.dot_general(
            h, w, (((1,), (1,)), ((), ())), preferred_element_type=jnp.float32
        )
        m = jnp.max(logits, axis=-1, keepdims=True)
        e = jnp.exp(logits - m)
        return e / jnp.sum(e, axis=-1, keepdims=True)

    for s in range(NSTREAMS):
        out_ref[s * BLOCK_M:(s + 1) * BLOCK_M, :] = probs(h_refs[s][...])


def kernel(hidden_states, gate_weight):
    n_tokens, hidden = hidden_states.shape
    n_experts = gate_weight.shape[0]
    grid = (n_tokens // (NSTREAMS * BLOCK_M),)
    h_specs = [
        pl.BlockSpec((BLOCK_M, hidden), lambda i, s=s: (i * NSTREAMS + s, 0))
        for s in range(NSTREAMS)
    ]
    return pl.pallas_call(
        _router_block,
        grid=grid,
        in_specs=h_specs + [pl.BlockSpec((n_experts, hidden), lambda i: (0, 0))],
        out_specs=pl.BlockSpec((NSTREAMS * BLOCK_M, n_experts), lambda i: (i, 0)),
        out_shape=jax.ShapeDtypeStruct((n_tokens, n_experts), jnp.float32),
        compiler_params=pltpu.CompilerParams(
            dimension_semantics=("arbitrary",),
        ),
    )(*([hidden_states] * NSTREAMS), gate_weight)
